# baseline (speedup 1.0000x reference)
import functools

import jax
import jax.numpy as jnp
from jax import lax
from jax.experimental import pallas as pl
from jax.experimental.pallas import tpu as pltpu
from jax.experimental.pallas import tpu_sc as plsc

N = 10000
NFEAT = 128
NPAD = 10240
E = 320000
NC = 2
NS = 16
NW = NC * NS
CHUNK = 128
NCH = 80
IB = 8
EPAD = NW * NCH * CHUNK
RPT = NPAD // NS
LEAK = 0.01

_MESH = plsc.VectorSubcoreMesh(core_axis_name="c", subcore_axis_name="s")


def _fill_zero(ref, nrows, ncols):
    vec = jnp.zeros((16,), jnp.float32)

    @pl.loop(0, nrows)
    def _(r):
        for k in range(ncols // 16):
            ref[r, pl.ds(k * 16, 16)] = vec


def _fill_zero1(ref, n):
    vec = jnp.zeros((16,), jnp.float32)
    for k in range(n // 16):
        ref[pl.ds(k * 16, 16)] = vec


TOTCH = NW * NCH
C0 = 0
C1 = 160


def _make_sc_prop(D):
    rank1 = D == 1
    tshape = (NPAD,) if rank1 else (NPAD, D)
    bufshape = (CHUNK,) if rank1 else (CHUNK, D)

    @functools.partial(
        pl.kernel,
        out_type=jax.ShapeDtypeStruct((NC,) + tshape, jnp.float32),
        mesh=_MESH,
        scratch_types=[
            pltpu.VMEM((2, IB, CHUNK), jnp.int32),
            pltpu.VMEM((2, IB, CHUNK), jnp.int32),
            pltpu.VMEM(bufshape, jnp.float32),
            pltpu.VMEM(bufshape, jnp.float32),
            pltpu.VMEM_SHARED(tshape, jnp.float32),
            pltpu.SemaphoreType.DMA,
            pltpu.SemaphoreType.DMA,
            pltpu.SemaphoreType.DMA,
            pltpu.SemaphoreType.DMA,
            pltpu.SemaphoreType.DMA,
            pltpu.SemaphoreType.DMA,
        ],
    )
    def prop(h_hbm, sd_hbm, out_hbm, sidx, didx, rows0, rows1, acc,
             gsem0, gsem1, ssem0, ssem1, isem0, isem1):
        c = lax.axis_index("c")
        s = lax.axis_index("s")
        base = s * RPT
        nch = C0 + c * (C1 - C0)
        nbh = (C0 // (2 * IB)) + c * ((C1 - C0) // (2 * IB))
        cb = c * (NS * C0) + s * nch
        rows = (rows0, rows1)
        gsem = (gsem0, gsem1)
        ssem = (ssem0, ssem1)
        isem = (isem0, isem1)

        def idx_start(b, q):
            pltpu.make_async_copy(sd_hbm.at[0, pl.ds(cb + b * IB, IB)],
                                  sidx.at[q], isem[q]).start()
            pltpu.make_async_copy(sd_hbm.at[1, pl.ds(cb + b * IB, IB)],
                                  didx.at[q], isem[q]).start()

        def idx_wait(q):
            pltpu.make_async_copy(sd_hbm.at[0, pl.ds(0, IB)],
                                  sidx.at[q], isem[q]).wait()
            pltpu.make_async_copy(sd_hbm.at[1, pl.ds(0, IB)],
                                  didx.at[q], isem[q]).wait()

        def g_start(q, j, p):
            pltpu.make_async_copy(h_hbm.at[sidx.at[q, j]], rows[p],
                                  gsem[p]).start()

        def g_wait(p):
            pltpu.make_async_copy(h_hbm.at[sidx.at[0, 0]], rows[p],
                                  gsem[p]).wait()

        def s_start(q, j, p):
            pltpu.make_async_copy(rows[p], acc.at[didx.at[q, j]],
                                  ssem[p]).start(add=True)

        def s_wait(p):
            pltpu.make_async_copy(rows[p], acc.at[didx.at[0, 0]],
                                  ssem[p]).wait()

        if rank1:
            _fill_zero1(rows0, CHUNK)
        else:
            _fill_zero(rows0, CHUNK, D)
        for k in range(RPT // CHUNK):
            pltpu.sync_copy(rows0, acc.at[pl.ds(base + k * CHUNK, CHUNK)])
        plsc.subcore_barrier()

        @pl.when(nch > 0)
        def _():
            pltpu.sync_copy(sd_hbm.at[0, pl.ds(cb, IB)], sidx.at[0])
            pltpu.sync_copy(sd_hbm.at[1, pl.ds(cb, IB)], didx.at[0])
            idx_start(1, 1)
            g_start(0, 0, 0)

        @pl.loop(0, nbh)
        def _(t):
            for half in range(2):
                q = half
                for j in range(IB):
                    p = j % 2
                    g_wait(p)
                    if j == 0:
                        if half == 0:
                            @pl.when(t > 0)
                            def _():
                                s_wait(1)
                                idx_start(2 * t + 1, 1)
                        else:
                            s_wait(1)

                            @pl.when(t < nbh - 1)
                            def _():
                                idx_start(2 * t + 2, 0)
                    else:
                        s_wait(1 - p)
                    if j < IB - 1:
                        g_start(q, j + 1, 1 - p)
                    elif half == 0:
                        idx_wait(1)
                        g_start(1, 0, 0)
                    else:
                        @pl.when(t < nbh - 1)
                        def _():
                            idx_wait(0)
                            g_start(0, 0, 0)
                    s_start(q, j, p)

        @pl.when(nch > 0)
        def _():
            s_wait(1)

        plsc.subcore_barrier()
        pltpu.sync_copy(acc.at[pl.ds(base, RPT)],
                        out_hbm.at[c, pl.ds(base, RPT)])

    return prop


_sc_prop128 = _make_sc_prop(128)
_sc_prop1 = _make_sc_prop(1)


BLK = 1280
_GRID = NPAD // BLK


def _rowmask(i, rows=BLK):
    r = i * rows + lax.broadcasted_iota(jnp.int32, (rows, 1), 0)
    return r < N


def _deg_to_scale(dref, i, rows=BLK):
    d = dref[0] + dref[1]
    return jnp.where(_rowmask(i, rows), lax.rsqrt(jnp.maximum(d, 1.0)), 0.0)


def _dot(x, w):
    return jnp.dot(x, w, preferred_element_type=jnp.float32,
                   precision=lax.Precision.HIGHEST)


def _deg_spec():
    return pl.BlockSpec((2, BLK, 1), lambda i: (0, i, 0))


def _feat_spec():
    return pl.BlockSpec((BLK, NFEAT), lambda i: (i, 0))


def _tc_linear(x, W, brow):
    def body(x_ref, w_ref, b_ref, o_ref):
        o_ref[...] = _dot(x_ref[...], w_ref[...]) + b_ref[...]

    return pl.pallas_call(
        body,
        out_shape=jax.ShapeDtypeStruct((NPAD, NFEAT), jnp.float32),
        grid=(_GRID,),
        in_specs=[_feat_spec(),
                  pl.BlockSpec((NFEAT, NFEAT), lambda i: (0, 0)),
                  pl.BlockSpec((1, NFEAT), lambda i: (0, 0))],
        out_specs=_feat_spec(),
    )(x, W, brow)


def _tc_scale(t, dsrc):
    def body(t_ref, ds_ref, o_ref):
        a = _deg_to_scale(ds_ref, pl.program_id(0))
        o_ref[...] = a * t_ref[...]

    return pl.pallas_call(
        body,
        out_shape=jax.ShapeDtypeStruct((NPAD, NFEAT), jnp.float32),
        grid=(_GRID,),
        in_specs=[_feat_spec(), _deg_spec()],
        out_specs=_feat_spec(),
    )(t, dsrc)


def _tc_combine1(p, dsrc, ddst, W, brow, gprev):
    def body(p_ref, ds_ref, dd_ref, w_ref, b_ref, g_ref, o_ref):
        i = pl.program_id(0)
        b = _deg_to_scale(dd_ref, i)
        a = _deg_to_scale(ds_ref, i)
        hpre = (p_ref[0] + p_ref[1]) * b
        h = jnp.where(hpre >= 0, hpre, LEAK * hpre)
        o_ref[...] = a * (_dot(h, w_ref[...]) + b_ref[...])

    return pl.pallas_call(
        body,
        out_shape=jax.ShapeDtypeStruct((NPAD, NFEAT), jnp.float32),
        grid=(_GRID,),
        in_specs=[pl.BlockSpec((2, BLK, NFEAT), lambda i: (0, i, 0)),
                  _deg_spec(), _deg_spec(),
                  pl.BlockSpec((NFEAT, NFEAT), lambda i: (0, 0)),
                  pl.BlockSpec((1, NFEAT), lambda i: (0, 0)),
                  _feat_spec()],
        out_specs=_feat_spec(),
        input_output_aliases={5: 0},
    )(p, dsrc, ddst, W, brow, gprev)


def _tc_combine2(p, dsrc, ddst, w2row, b2s):
    def body(p_ref, ds_ref, dd_ref, w_ref, b_ref, h_ref, g_ref):
        i = pl.program_id(0)
        b = _deg_to_scale(dd_ref, i)
        a = _deg_to_scale(ds_ref, i)
        hpre = (p_ref[0] + p_ref[1]) * b
        h = jnp.where(hpre >= 0, hpre, LEAK * hpre)
        h_ref[...] = h
        sv = jnp.sum(h * w_ref[...], axis=1, keepdims=True) + b_ref[0, 0]
        g_ref[...] = a * sv

    return pl.pallas_call(
        body,
        out_shape=(jax.ShapeDtypeStruct((NPAD, NFEAT), jnp.float32),
                   jax.ShapeDtypeStruct((NPAD, 1), jnp.float32)),
        grid=(_GRID,),
        in_specs=[pl.BlockSpec((2, BLK, NFEAT), lambda i: (0, i, 0)),
                  _deg_spec(), _deg_spec(),
                  pl.BlockSpec((1, NFEAT), lambda i: (0, 0)),
                  pl.BlockSpec((1, 1), lambda i: (0, 0))],
        out_specs=(_feat_spec(),
                   pl.BlockSpec((BLK, 1), lambda i: (i, 0))),
    )(p, dsrc, ddst, w2row, b2s)


def _tc_softmax(p, ddst):
    def body(p_ref, dd_ref, o_ref):
        b = _deg_to_scale(dd_ref, 0, NPAD)
        l = b * (p_ref[0] + p_ref[1])
        mask = _rowmask(0, NPAD)
        lm = jnp.where(mask, l, -1e30)
        m = jnp.max(lm)
        e = jnp.where(mask, jnp.exp(lm - m), 0.0)
        o_ref[...] = e / jnp.sum(e)

    return pl.pallas_call(
        body,
        out_shape=jax.ShapeDtypeStruct((NPAD, 1), jnp.float32),
        grid=(1,),
        in_specs=[pl.BlockSpec((2, NPAD, 1), lambda i: (0, 0, 0)),
                  pl.BlockSpec((2, NPAD, 1), lambda i: (0, 0, 0))],
        out_specs=pl.BlockSpec((NPAD, 1), lambda i: (0, 0)),
    )(p, ddst)


def kernel(x, adj, W0, b0, W1, b1, W2, b2):
    src = adj[0].astype(jnp.int32)
    dst = adj[1].astype(jnp.int32)
    pad = jnp.full((EPAD - E,), N, jnp.int32)
    sp = jnp.concatenate([src, pad]).reshape(TOTCH, CHUNK)
    dp = jnp.concatenate([dst, pad]).reshape(TOTCH, CHUNK)
    sd3 = jnp.stack([sp, dp])
    ds3 = jnp.stack([dp, sp])
    xp = jnp.zeros((NPAD, NFEAT), jnp.float32).at[:N].set(x)
    ones1 = jnp.ones((NPAD,), jnp.float32)

    dsrc = _sc_prop1(ones1, ds3).reshape(NC, NPAD, 1)
    t0 = _tc_linear(xp, W0, b0.reshape(1, -1))
    ddst = _sc_prop1(ones1, sd3).reshape(NC, NPAD, 1)
    g0 = _tc_scale(t0, dsrc)
    p1 = _sc_prop128(g0, sd3)
    g1 = _tc_combine1(p1, dsrc, ddst, W1, b1.reshape(1, -1), g0)
    p2 = _sc_prop128(g1, sd3)
    h2, g2 = _tc_combine2(p2, dsrc, ddst, W2.reshape(1, -1), b2.reshape(1, 1))
    p3 = _sc_prop1(g2.reshape(NPAD), sd3).reshape(NC, NPAD, 1)
    w = _tc_softmax(p3, ddst)
    return (w[:N], h2[:N])


# speedup vs baseline: 1.3930x; 1.3930x over previous
import functools

import jax
import jax.numpy as jnp
from jax import lax
from jax.experimental import pallas as pl
from jax.experimental.pallas import tpu as pltpu
from jax.experimental.pallas import tpu_sc as plsc

N = 10000
NFEAT = 128
NPAD = 10240
E = 320000
NC = 2
NS = 16
NW = NC * NS
CHUNK = 64
NCH = 160
IB = 8
NBUF = 4
EPAD = NW * NCH * CHUNK
RPT = NPAD // NS
LEAK = 0.01

_MESH = plsc.VectorSubcoreMesh(core_axis_name="c", subcore_axis_name="s")


def _fill_zero(ref, nrows, ncols):
    vec = jnp.zeros((16,), jnp.float32)

    @pl.loop(0, nrows)
    def _(r):
        for k in range(ncols // 16):
            ref[r, pl.ds(k * 16, 16)] = vec


def _fill_zero1(ref, n):
    vec = jnp.zeros((16,), jnp.float32)
    for k in range(n // 16):
        ref[pl.ds(k * 16, 16)] = vec


TOTCH = NW * NCH
C0 = NCH
C1 = NCH


def _make_sc_prop(D):
    rank1 = D == 1
    tshape = (NPAD,) if rank1 else (NPAD, D)
    bufshape = (CHUNK,) if rank1 else (CHUNK, D)

    @functools.partial(
        pl.kernel,
        out_type=jax.ShapeDtypeStruct((NC,) + tshape, jnp.float32),
        mesh=_MESH,
        scratch_types=[
            pltpu.VMEM((2, IB, CHUNK), jnp.int32),
            pltpu.VMEM((2, IB, CHUNK), jnp.int32),
            pltpu.VMEM(bufshape, jnp.float32),
            pltpu.VMEM(bufshape, jnp.float32),
            pltpu.VMEM(bufshape, jnp.float32),
            pltpu.VMEM(bufshape, jnp.float32),
            pltpu.VMEM_SHARED(tshape, jnp.float32),
            pltpu.SemaphoreType.DMA,
            pltpu.SemaphoreType.DMA,
            pltpu.SemaphoreType.DMA,
            pltpu.SemaphoreType.DMA,
            pltpu.SemaphoreType.DMA,
            pltpu.SemaphoreType.DMA,
            pltpu.SemaphoreType.DMA,
            pltpu.SemaphoreType.DMA,
            pltpu.SemaphoreType.DMA,
            pltpu.SemaphoreType.DMA,
        ],
    )
    def prop(h_hbm, sd_hbm, out_hbm, sidx, didx, r0, r1, r2, r3, acc,
             g0, g1, g2, g3, s0, s1, s2, s3, isem0, isem1):
        c = lax.axis_index("c")
        s = lax.axis_index("s")
        base = s * RPT
        nch = C0 + c * (C1 - C0)
        nbh = (C0 // (2 * IB)) + c * ((C1 - C0) // (2 * IB))
        cb = c * (NS * C0) + s * nch
        rows = (r0, r1, r2, r3)
        gsem = (g0, g1, g2, g3)
        ssem = (s0, s1, s2, s3)
        isem = (isem0, isem1)

        def idx_start(b, q):
            pltpu.make_async_copy(sd_hbm.at[0, pl.ds(cb + b * IB, IB)],
                                  sidx.at[q], isem[q]).start()
            pltpu.make_async_copy(sd_hbm.at[1, pl.ds(cb + b * IB, IB)],
                                  didx.at[q], isem[q]).start()

        def idx_wait(q):
            pltpu.make_async_copy(sd_hbm.at[0, pl.ds(0, IB)],
                                  sidx.at[q], isem[q]).wait()
            pltpu.make_async_copy(sd_hbm.at[1, pl.ds(0, IB)],
                                  didx.at[q], isem[q]).wait()

        def g_start(q, j, p):
            pltpu.make_async_copy(h_hbm.at[sidx.at[q, j]], rows[p],
                                  gsem[p]).start()

        def g_wait(p):
            pltpu.make_async_copy(h_hbm.at[sidx.at[0, 0]], rows[p],
                                  gsem[p]).wait()

        def s_start(q, j, p):
            pltpu.make_async_copy(rows[p], acc.at[didx.at[q, j]],
                                  ssem[p]).start(add=True)

        def s_wait(p):
            pltpu.make_async_copy(rows[p], acc.at[didx.at[0, 0]],
                                  ssem[p]).wait()

        if rank1:
            _fill_zero1(r0, CHUNK)
        else:
            _fill_zero(r0, CHUNK, D)
        for k in range(RPT // CHUNK):
            pltpu.sync_copy(r0, acc.at[pl.ds(base + k * CHUNK, CHUNK)])
        plsc.subcore_barrier()

        @pl.when(nch > 0)
        def _():
            pltpu.sync_copy(sd_hbm.at[0, pl.ds(cb, IB)], sidx.at[0])
            pltpu.sync_copy(sd_hbm.at[1, pl.ds(cb, IB)], didx.at[0])
            idx_start(1, 1)
            g_start(0, 0, 0)
            g_start(0, 1, 1)

        @pl.loop(0, nbh)
        def _(t):
            for half in range(2):
                q = half
                for j in range(IB):
                    p = j % NBUF
                    np_ = (j + 2) % NBUF
                    g_wait(p)
                    if half == 0 and j < 2:
                        @pl.when(t > 0)
                        def _():
                            s_wait(np_)
                            if j == 1:
                                idx_start(2 * t + 1, 1)
                    else:
                        s_wait(np_)
                        if half == 1 and j == 1:
                            @pl.when(t < nbh - 1)
                            def _():
                                idx_start(2 * t + 2, 0)
                    if j == IB - 2:
                        if half == 0:
                            idx_wait(1)
                            g_start(1, 0, np_)
                        else:
                            @pl.when(t < nbh - 1)
                            def _():
                                idx_wait(0)
                                g_start(0, 0, np_)
                    elif j == IB - 1:
                        if half == 0:
                            g_start(1, 1, np_)
                        else:
                            @pl.when(t < nbh - 1)
                            def _():
                                g_start(0, 1, np_)
                    else:
                        g_start(q, j + 2, np_)
                    s_start(q, j, p)

        @pl.when(nch > 0)
        def _():
            s_wait((NCH - 2) % NBUF)
            s_wait((NCH - 1) % NBUF)

        plsc.subcore_barrier()
        pltpu.sync_copy(acc.at[pl.ds(base, RPT)],
                        out_hbm.at[c, pl.ds(base, RPT)])

    return prop


_sc_prop128 = _make_sc_prop(128)
_sc_prop1 = _make_sc_prop(1)


BLK = 1280
_GRID = NPAD // BLK


def _rowmask(i, rows=BLK):
    r = i * rows + lax.broadcasted_iota(jnp.int32, (rows, 1), 0)
    return r < N


def _deg_to_scale(dref, i, rows=BLK):
    d = dref[0] + dref[1]
    return jnp.where(_rowmask(i, rows), lax.rsqrt(jnp.maximum(d, 1.0)), 0.0)


def _dot(x, w):
    return jnp.dot(x, w, preferred_element_type=jnp.float32,
                   precision=lax.Precision.HIGHEST)


def _deg_spec():
    return pl.BlockSpec((2, BLK, 1), lambda i: (0, i, 0))


def _feat_spec():
    return pl.BlockSpec((BLK, NFEAT), lambda i: (i, 0))


def _tc_linear(x, W, brow):
    def body(x_ref, w_ref, b_ref, o_ref):
        o_ref[...] = _dot(x_ref[...], w_ref[...]) + b_ref[...]

    return pl.pallas_call(
        body,
        out_shape=jax.ShapeDtypeStruct((NPAD, NFEAT), jnp.float32),
        grid=(_GRID,),
        in_specs=[_feat_spec(),
                  pl.BlockSpec((NFEAT, NFEAT), lambda i: (0, 0)),
                  pl.BlockSpec((1, NFEAT), lambda i: (0, 0))],
        out_specs=_feat_spec(),
    )(x, W, brow)


def _tc_scale(t, dsrc):
    def body(t_ref, ds_ref, o_ref):
        a = _deg_to_scale(ds_ref, pl.program_id(0))
        o_ref[...] = a * t_ref[...]

    return pl.pallas_call(
        body,
        out_shape=jax.ShapeDtypeStruct((NPAD, NFEAT), jnp.float32),
        grid=(_GRID,),
        in_specs=[_feat_spec(), _deg_spec()],
        out_specs=_feat_spec(),
    )(t, dsrc)


def _tc_combine1(p, dsrc, ddst, W, brow, gprev):
    def body(p_ref, ds_ref, dd_ref, w_ref, b_ref, g_ref, o_ref):
        i = pl.program_id(0)
        b = _deg_to_scale(dd_ref, i)
        a = _deg_to_scale(ds_ref, i)
        hpre = (p_ref[0] + p_ref[1]) * b
        h = jnp.where(hpre >= 0, hpre, LEAK * hpre)
        o_ref[...] = a * (_dot(h, w_ref[...]) + b_ref[...])

    return pl.pallas_call(
        body,
        out_shape=jax.ShapeDtypeStruct((NPAD, NFEAT), jnp.float32),
        grid=(_GRID,),
        in_specs=[pl.BlockSpec((2, BLK, NFEAT), lambda i: (0, i, 0)),
                  _deg_spec(), _deg_spec(),
                  pl.BlockSpec((NFEAT, NFEAT), lambda i: (0, 0)),
                  pl.BlockSpec((1, NFEAT), lambda i: (0, 0)),
                  _feat_spec()],
        out_specs=_feat_spec(),
        input_output_aliases={5: 0},
    )(p, dsrc, ddst, W, brow, gprev)


def _tc_combine2(p, dsrc, ddst, w2row, b2s):
    def body(p_ref, ds_ref, dd_ref, w_ref, b_ref, h_ref, g_ref):
        i = pl.program_id(0)
        b = _deg_to_scale(dd_ref, i)
        a = _deg_to_scale(ds_ref, i)
        hpre = (p_ref[0] + p_ref[1]) * b
        h = jnp.where(hpre >= 0, hpre, LEAK * hpre)
        h_ref[...] = h
        sv = jnp.sum(h * w_ref[...], axis=1, keepdims=True) + b_ref[0, 0]
        g_ref[...] = a * sv

    return pl.pallas_call(
        body,
        out_shape=(jax.ShapeDtypeStruct((NPAD, NFEAT), jnp.float32),
                   jax.ShapeDtypeStruct((NPAD, 1), jnp.float32)),
        grid=(_GRID,),
        in_specs=[pl.BlockSpec((2, BLK, NFEAT), lambda i: (0, i, 0)),
                  _deg_spec(), _deg_spec(),
                  pl.BlockSpec((1, NFEAT), lambda i: (0, 0)),
                  pl.BlockSpec((1, 1), lambda i: (0, 0))],
        out_specs=(_feat_spec(),
                   pl.BlockSpec((BLK, 1), lambda i: (i, 0))),
    )(p, dsrc, ddst, w2row, b2s)


def _tc_softmax(p, ddst):
    def body(p_ref, dd_ref, o_ref):
        b = _deg_to_scale(dd_ref, 0, NPAD)
        l = b * (p_ref[0] + p_ref[1])
        mask = _rowmask(0, NPAD)
        lm = jnp.where(mask, l, -1e30)
        m = jnp.max(lm)
        e = jnp.where(mask, jnp.exp(lm - m), 0.0)
        o_ref[...] = e / jnp.sum(e)

    return pl.pallas_call(
        body,
        out_shape=jax.ShapeDtypeStruct((NPAD, 1), jnp.float32),
        grid=(1,),
        in_specs=[pl.BlockSpec((2, NPAD, 1), lambda i: (0, 0, 0)),
                  pl.BlockSpec((2, NPAD, 1), lambda i: (0, 0, 0))],
        out_specs=pl.BlockSpec((NPAD, 1), lambda i: (0, 0)),
    )(p, ddst)


def kernel(x, adj, W0, b0, W1, b1, W2, b2):
    src = adj[0].astype(jnp.int32)
    dst = adj[1].astype(jnp.int32)
    pad = jnp.full((EPAD - E,), N, jnp.int32)
    sp = jnp.concatenate([src, pad]).reshape(TOTCH, CHUNK)
    dp = jnp.concatenate([dst, pad]).reshape(TOTCH, CHUNK)
    sd3 = jnp.stack([sp, dp])
    ds3 = jnp.stack([dp, sp])
    xp = jnp.zeros((NPAD, NFEAT), jnp.float32).at[:N].set(x)
    ones1 = jnp.ones((NPAD,), jnp.float32)

    dsrc = _sc_prop1(ones1, ds3).reshape(NC, NPAD, 1)
    t0 = _tc_linear(xp, W0, b0.reshape(1, -1))
    ddst = _sc_prop1(ones1, sd3).reshape(NC, NPAD, 1)
    g0 = _tc_scale(t0, dsrc)
    p1 = _sc_prop128(g0, sd3)
    g1 = _tc_combine1(p1, dsrc, ddst, W1, b1.reshape(1, -1), g0)
    p2 = _sc_prop128(g1, sd3)
    h2, g2 = _tc_combine2(p2, dsrc, ddst, W2.reshape(1, -1), b2.reshape(1, 1))
    p3 = _sc_prop1(g2.reshape(NPAD), sd3).reshape(NC, NPAD, 1)
    w = _tc_softmax(p3, ddst)
    return (w[:N], h2[:N])


# speedup vs baseline: 1.4004x; 1.0053x over previous
import functools

import jax
import jax.numpy as jnp
from jax import lax
from jax.experimental import pallas as pl
from jax.experimental.pallas import tpu as pltpu
from jax.experimental.pallas import tpu_sc as plsc

N = 10000
NFEAT = 128
NPAD = 10240
E = 320000
NC = 2
NS = 16
NW = NC * NS
CHUNK = 64
NCH = 160
IB = 8
NBUF = 4
EPAD = NW * NCH * CHUNK
RPT = NPAD // NS
LEAK = 0.01

_MESH = plsc.VectorSubcoreMesh(core_axis_name="c", subcore_axis_name="s")


def _fill_zero(ref, nrows, ncols):
    vec = jnp.zeros((16,), jnp.float32)

    @pl.loop(0, nrows)
    def _(r):
        for k in range(ncols // 16):
            ref[r, pl.ds(k * 16, 16)] = vec


def _fill_zero1(ref, n):
    vec = jnp.zeros((16,), jnp.float32)
    for k in range(n // 16):
        ref[pl.ds(k * 16, 16)] = vec


TOTCH = NW * NCH
C0 = NCH
C1 = NCH


def _make_sc_prop(D):
    rank1 = D == 1
    tshape = (NPAD,) if rank1 else (NPAD, D)
    bufshape = (CHUNK,) if rank1 else (CHUNK, D)

    @functools.partial(
        pl.kernel,
        out_type=jax.ShapeDtypeStruct((NC,) + tshape, jnp.float32),
        mesh=_MESH,
        scratch_types=[
            pltpu.VMEM((2, IB, CHUNK), jnp.int32),
            pltpu.VMEM((2, IB, CHUNK), jnp.int32),
            pltpu.VMEM(bufshape, jnp.float32),
            pltpu.VMEM(bufshape, jnp.float32),
            pltpu.VMEM(bufshape, jnp.float32),
            pltpu.VMEM(bufshape, jnp.float32),
            pltpu.VMEM_SHARED(tshape, jnp.float32),
            pltpu.SemaphoreType.DMA,
            pltpu.SemaphoreType.DMA,
            pltpu.SemaphoreType.DMA,
            pltpu.SemaphoreType.DMA,
            pltpu.SemaphoreType.DMA,
            pltpu.SemaphoreType.DMA,
            pltpu.SemaphoreType.DMA,
            pltpu.SemaphoreType.DMA,
            pltpu.SemaphoreType.DMA,
            pltpu.SemaphoreType.DMA,
        ],
    )
    def prop(h_hbm, sd_hbm, out_hbm, sidx, didx, r0, r1, r2, r3, acc,
             g0, g1, g2, g3, s0, s1, s2, s3, isem0, isem1):
        c = lax.axis_index("c")
        s = lax.axis_index("s")
        base = s * RPT
        nch = C0 + c * (C1 - C0)
        nbh = (C0 // (2 * IB)) + c * ((C1 - C0) // (2 * IB))
        cb = c * (NS * C0) + s * nch
        rows = (r0, r1, r2, r3)
        gsem = (g0, g1, g2, g3)
        ssem = (s0, s1, s2, s3)
        isem = (isem0, isem1)

        def idx_start(b, q):
            pltpu.make_async_copy(sd_hbm.at[0, pl.ds(cb + b * IB, IB)],
                                  sidx.at[q], isem[q]).start()
            pltpu.make_async_copy(sd_hbm.at[1, pl.ds(cb + b * IB, IB)],
                                  didx.at[q], isem[q]).start()

        def idx_wait(q):
            pltpu.make_async_copy(sd_hbm.at[0, pl.ds(0, IB)],
                                  sidx.at[q], isem[q]).wait()
            pltpu.make_async_copy(sd_hbm.at[1, pl.ds(0, IB)],
                                  didx.at[q], isem[q]).wait()

        def g_start(q, j, p):
            pltpu.make_async_copy(h_hbm.at[sidx.at[q, j]], rows[p],
                                  gsem[p]).start()

        def g_wait(p):
            pltpu.make_async_copy(h_hbm.at[sidx.at[0, 0]], rows[p],
                                  gsem[p]).wait()

        def s_start(q, j, p):
            pass

        def s_wait(p):
            pass

        if rank1:
            _fill_zero1(r0, CHUNK)
        else:
            _fill_zero(r0, CHUNK, D)
        for k in range(RPT // CHUNK):
            pltpu.sync_copy(r0, acc.at[pl.ds(base + k * CHUNK, CHUNK)])
        plsc.subcore_barrier()

        @pl.when(nch > 0)
        def _():
            pltpu.sync_copy(sd_hbm.at[0, pl.ds(cb, IB)], sidx.at[0])
            pltpu.sync_copy(sd_hbm.at[1, pl.ds(cb, IB)], didx.at[0])
            idx_start(1, 1)
            g_start(0, 0, 0)
            g_start(0, 1, 1)

        @pl.loop(0, nbh)
        def _(t):
            for half in range(2):
                q = half
                for j in range(IB):
                    p = j % NBUF
                    np_ = (j + 2) % NBUF
                    g_wait(p)
                    if half == 0 and j < 2:
                        @pl.when(t > 0)
                        def _():
                            s_wait(np_)
                            if j == 1:
                                idx_start(2 * t + 1, 1)
                    else:
                        s_wait(np_)
                        if half == 1 and j == 1:
                            @pl.when(t < nbh - 1)
                            def _():
                                idx_start(2 * t + 2, 0)
                    if j == IB - 2:
                        if half == 0:
                            idx_wait(1)
                            g_start(1, 0, np_)
                        else:
                            @pl.when(t < nbh - 1)
                            def _():
                                idx_wait(0)
                                g_start(0, 0, np_)
                    elif j == IB - 1:
                        if half == 0:
                            g_start(1, 1, np_)
                        else:
                            @pl.when(t < nbh - 1)
                            def _():
                                g_start(0, 1, np_)
                    else:
                        g_start(q, j + 2, np_)
                    s_start(q, j, p)

        @pl.when(nch > 0)
        def _():
            s_wait((NCH - 2) % NBUF)
            s_wait((NCH - 1) % NBUF)

        plsc.subcore_barrier()
        pltpu.sync_copy(acc.at[pl.ds(base, RPT)],
                        out_hbm.at[c, pl.ds(base, RPT)])

    return prop


_sc_prop128 = _make_sc_prop(128)
_sc_prop1 = _make_sc_prop(1)


BLK = 1280
_GRID = NPAD // BLK


def _rowmask(i, rows=BLK):
    r = i * rows + lax.broadcasted_iota(jnp.int32, (rows, 1), 0)
    return r < N


def _deg_to_scale(dref, i, rows=BLK):
    d = dref[0] + dref[1]
    return jnp.where(_rowmask(i, rows), lax.rsqrt(jnp.maximum(d, 1.0)), 0.0)


def _dot(x, w):
    return jnp.dot(x, w, preferred_element_type=jnp.float32,
                   precision=lax.Precision.HIGHEST)


def _deg_spec():
    return pl.BlockSpec((2, BLK, 1), lambda i: (0, i, 0))


def _feat_spec():
    return pl.BlockSpec((BLK, NFEAT), lambda i: (i, 0))


def _tc_linear(x, W, brow):
    def body(x_ref, w_ref, b_ref, o_ref):
        o_ref[...] = _dot(x_ref[...], w_ref[...]) + b_ref[...]

    return pl.pallas_call(
        body,
        out_shape=jax.ShapeDtypeStruct((NPAD, NFEAT), jnp.float32),
        grid=(_GRID,),
        in_specs=[_feat_spec(),
                  pl.BlockSpec((NFEAT, NFEAT), lambda i: (0, 0)),
                  pl.BlockSpec((1, NFEAT), lambda i: (0, 0))],
        out_specs=_feat_spec(),
    )(x, W, brow)


def _tc_scale(t, dsrc):
    def body(t_ref, ds_ref, o_ref):
        a = _deg_to_scale(ds_ref, pl.program_id(0))
        o_ref[...] = a * t_ref[...]

    return pl.pallas_call(
        body,
        out_shape=jax.ShapeDtypeStruct((NPAD, NFEAT), jnp.float32),
        grid=(_GRID,),
        in_specs=[_feat_spec(), _deg_spec()],
        out_specs=_feat_spec(),
    )(t, dsrc)


def _tc_combine1(p, dsrc, ddst, W, brow, gprev):
    def body(p_ref, ds_ref, dd_ref, w_ref, b_ref, g_ref, o_ref):
        i = pl.program_id(0)
        b = _deg_to_scale(dd_ref, i)
        a = _deg_to_scale(ds_ref, i)
        hpre = (p_ref[0] + p_ref[1]) * b
        h = jnp.where(hpre >= 0, hpre, LEAK * hpre)
        o_ref[...] = a * (_dot(h, w_ref[...]) + b_ref[...])

    return pl.pallas_call(
        body,
        out_shape=jax.ShapeDtypeStruct((NPAD, NFEAT), jnp.float32),
        grid=(_GRID,),
        in_specs=[pl.BlockSpec((2, BLK, NFEAT), lambda i: (0, i, 0)),
                  _deg_spec(), _deg_spec(),
                  pl.BlockSpec((NFEAT, NFEAT), lambda i: (0, 0)),
                  pl.BlockSpec((1, NFEAT), lambda i: (0, 0)),
                  _feat_spec()],
        out_specs=_feat_spec(),
        input_output_aliases={5: 0},
    )(p, dsrc, ddst, W, brow, gprev)


def _tc_combine2(p, dsrc, ddst, w2row, b2s):
    def body(p_ref, ds_ref, dd_ref, w_ref, b_ref, h_ref, g_ref):
        i = pl.program_id(0)
        b = _deg_to_scale(dd_ref, i)
        a = _deg_to_scale(ds_ref, i)
        hpre = (p_ref[0] + p_ref[1]) * b
        h = jnp.where(hpre >= 0, hpre, LEAK * hpre)
        h_ref[...] = h
        sv = jnp.sum(h * w_ref[...], axis=1, keepdims=True) + b_ref[0, 0]
        g_ref[...] = a * sv

    return pl.pallas_call(
        body,
        out_shape=(jax.ShapeDtypeStruct((NPAD, NFEAT), jnp.float32),
                   jax.ShapeDtypeStruct((NPAD, 1), jnp.float32)),
        grid=(_GRID,),
        in_specs=[pl.BlockSpec((2, BLK, NFEAT), lambda i: (0, i, 0)),
                  _deg_spec(), _deg_spec(),
                  pl.BlockSpec((1, NFEAT), lambda i: (0, 0)),
                  pl.BlockSpec((1, 1), lambda i: (0, 0))],
        out_specs=(_feat_spec(),
                   pl.BlockSpec((BLK, 1), lambda i: (i, 0))),
    )(p, dsrc, ddst, w2row, b2s)


def _tc_softmax(p, ddst):
    def body(p_ref, dd_ref, o_ref):
        b = _deg_to_scale(dd_ref, 0, NPAD)
        l = b * (p_ref[0] + p_ref[1])
        mask = _rowmask(0, NPAD)
        lm = jnp.where(mask, l, -1e30)
        m = jnp.max(lm)
        e = jnp.where(mask, jnp.exp(lm - m), 0.0)
        o_ref[...] = e / jnp.sum(e)

    return pl.pallas_call(
        body,
        out_shape=jax.ShapeDtypeStruct((NPAD, 1), jnp.float32),
        grid=(1,),
        in_specs=[pl.BlockSpec((2, NPAD, 1), lambda i: (0, 0, 0)),
                  pl.BlockSpec((2, NPAD, 1), lambda i: (0, 0, 0))],
        out_specs=pl.BlockSpec((NPAD, 1), lambda i: (0, 0)),
    )(p, ddst)


def kernel(x, adj, W0, b0, W1, b1, W2, b2):
    src = adj[0].astype(jnp.int32)
    dst = adj[1].astype(jnp.int32)
    pad = jnp.full((EPAD - E,), N, jnp.int32)
    sp = jnp.concatenate([src, pad]).reshape(TOTCH, CHUNK)
    dp = jnp.concatenate([dst, pad]).reshape(TOTCH, CHUNK)
    sd3 = jnp.stack([sp, dp])
    ds3 = jnp.stack([dp, sp])
    xp = jnp.zeros((NPAD, NFEAT), jnp.float32).at[:N].set(x)
    ones1 = jnp.ones((NPAD,), jnp.float32)

    dsrc = _sc_prop1(ones1, ds3).reshape(NC, NPAD, 1)
    t0 = _tc_linear(xp, W0, b0.reshape(1, -1))
    ddst = _sc_prop1(ones1, sd3).reshape(NC, NPAD, 1)
    g0 = _tc_scale(t0, dsrc)
    p1 = _sc_prop128(g0, sd3)
    g1 = _tc_combine1(p1, dsrc, ddst, W1, b1.reshape(1, -1), g0)
    p2 = _sc_prop128(g1, sd3)
    h2, g2 = _tc_combine2(p2, dsrc, ddst, W2.reshape(1, -1), b2.reshape(1, 1))
    p3 = _sc_prop1(g2.reshape(NPAD), sd3).reshape(NC, NPAD, 1)
    w = _tc_softmax(p3, ddst)
    return (w[:N], h2[:N])


# speedup vs baseline: 3.0781x; 2.1980x over previous
import functools

import jax
import jax.numpy as jnp
from jax import lax
from jax.experimental import pallas as pl
from jax.experimental.pallas import tpu as pltpu
from jax.experimental.pallas import tpu_sc as plsc

N = 10000
NFEAT = 128
NPAD = 10240
E = 320000
NC = 2
NS = 16
NW = NC * NS
CHUNK = 128
IB = 8
NBUF = 4
EPAD = 327680
TOTCH = EPAD // CHUNK
RPT = NPAD // NS
LEAK = 0.01
DHALF = NFEAT // 2

_MESH = plsc.VectorSubcoreMesh(core_axis_name="c", subcore_axis_name="s")


def _fill_zero(ref, nrows, ncols):
    vec = jnp.zeros((16,), jnp.float32)

    @pl.loop(0, nrows)
    def _(r):
        for k in range(ncols // 16):
            ref[r, pl.ds(k * 16, 16)] = vec


def _fill_zero1(ref, n):
    vec = jnp.zeros((16,), jnp.float32)
    for k in range(n // 16):
        ref[pl.ds(k * 16, 16)] = vec


def _make_sc_prop(rank1):
    nbuf = 4 if rank1 else 2
    bufshape = (CHUNK,) if rank1 else (CHUNK, DHALF)
    nch = TOTCH // NW if rank1 else TOTCH // NS
    nbh = nch // (2 * IB)

    scratch = [
        pltpu.VMEM((2, IB, CHUNK), jnp.int32),
        pltpu.VMEM((2, IB, CHUNK), jnp.int32),
    ]
    scratch += [pltpu.VMEM(bufshape, jnp.float32)] * nbuf
    if rank1:
        scratch += [pltpu.VMEM_SHARED((NPAD,), jnp.float32)]
    else:
        scratch += [pltpu.VMEM_SHARED((NPAD, DHALF), jnp.float32),
                    pltpu.VMEM_SHARED((NPAD, DHALF), jnp.float32)]
    scratch += [pltpu.SemaphoreType.DMA] * (2 * nbuf + 2)

    if rank1:
        out_type = jax.ShapeDtypeStruct((NC, NPAD), jnp.float32)
        cp = None
    else:
        out_type = jax.ShapeDtypeStruct((NPAD, NFEAT), jnp.float32)
        cp = pltpu.CompilerParams(use_tc_tiling_on_sc=False)

    @functools.partial(pl.kernel, out_type=out_type, mesh=_MESH,
                       scratch_types=scratch, compiler_params=cp)
    def prop(*refs):
        if rank1:
            (h_hbm, sd_hbm, out_hbm, sidx, didx, r0, r1, r2, r3, acc,
             g0, g1, g2, g3, s0, s1, s2, s3, isem0, isem1) = refs
            rows, gsem, ssem = (r0, r1, r2, r3), (g0, g1, g2, g3), (s0, s1, s2, s3)
            isem = (isem0, isem1)
            tab = h_hbm
        else:
            (h_hbm, sd_hbm, out_hbm, sidx, didx,
             r0, r1, tab, acc, g0, g1, s0, s1, isem0, isem1) = refs
            rows, gsem, ssem = (r0, r1), (g0, g1), (s0, s1)
            isem = (isem0, isem1)

        c = lax.axis_index("c")
        s = lax.axis_index("s")
        base = s * RPT
        cb = ((c * NS + s) if rank1 else s) * nch

        def idx_start(b, q):
            pltpu.make_async_copy(sd_hbm.at[0, pl.ds(cb + b * IB, IB)],
                                  sidx.at[q], isem[q]).start()
            pltpu.make_async_copy(sd_hbm.at[1, pl.ds(cb + b * IB, IB)],
                                  didx.at[q], isem[q]).start()

        def idx_wait(q):
            pltpu.make_async_copy(sd_hbm.at[0, pl.ds(0, IB)],
                                  sidx.at[q], isem[q]).wait()
            pltpu.make_async_copy(sd_hbm.at[1, pl.ds(0, IB)],
                                  didx.at[q], isem[q]).wait()

        def g_start(q, j, p):
            pltpu.make_async_copy(tab.at[sidx.at[q, j]], rows[p],
                                  gsem[p]).start()

        def g_wait(p):
            pltpu.make_async_copy(tab.at[sidx.at[0, 0]], rows[p],
                                  gsem[p]).wait()

        def s_start(q, j, p):
            pltpu.make_async_copy(rows[p], acc.at[didx.at[q, j]],
                                  ssem[p]).start(add=True)

        def s_wait(p):
            pltpu.make_async_copy(rows[p], acc.at[didx.at[0, 0]],
                                  ssem[p]).wait()

        if rank1:
            _fill_zero1(r0, CHUNK)
        else:
            pltpu.sync_copy(
                h_hbm.at[pl.ds(base, RPT), pl.ds(c * DHALF, DHALF)],
                tab.at[pl.ds(base, RPT)])
            _fill_zero(r0, CHUNK, DHALF)
        for k in range(RPT // CHUNK):
            pltpu.sync_copy(r0, acc.at[pl.ds(base + k * CHUNK, CHUNK)])
        plsc.subcore_barrier()

        pltpu.sync_copy(sd_hbm.at[0, pl.ds(cb, IB)], sidx.at[0])
        pltpu.sync_copy(sd_hbm.at[1, pl.ds(cb, IB)], didx.at[0])
        idx_start(1, 1)
        g_start(0, 0, 0)
        if rank1:
            g_start(0, 1, 1)

        if rank1:
            @pl.loop(0, nbh)
            def _(t):
                for half in range(2):
                    q = half
                    for j in range(IB):
                        p = j % nbuf
                        np_ = (j + 2) % nbuf
                        g_wait(p)
                        if half == 0 and j < 2:
                            @pl.when(t > 0)
                            def _():
                                s_wait(np_)
                                if j == 1:
                                    idx_start(2 * t + 1, 1)
                        else:
                            s_wait(np_)
                            if half == 1 and j == 1:
                                @pl.when(t < nbh - 1)
                                def _():
                                    idx_start(2 * t + 2, 0)
                        if j == IB - 2:
                            if half == 0:
                                idx_wait(1)
                                g_start(1, 0, np_)
                            else:
                                @pl.when(t < nbh - 1)
                                def _():
                                    idx_wait(0)
                                    g_start(0, 0, np_)
                        elif j == IB - 1:
                            if half == 0:
                                g_start(1, 1, np_)
                            else:
                                @pl.when(t < nbh - 1)
                                def _():
                                    g_start(0, 1, np_)
                        else:
                            g_start(q, j + 2, np_)
                        s_start(q, j, p)

            s_wait((nch - 2) % nbuf)
            s_wait((nch - 1) % nbuf)
        else:
            @pl.loop(0, nbh)
            def _(t):
                for half in range(2):
                    q = half
                    for j in range(IB):
                        p = j % 2
                        g_wait(p)
                        if j == 0:
                            if half == 0:
                                @pl.when(t > 0)
                                def _():
                                    s_wait(1)
                                    idx_start(2 * t + 1, 1)
                            else:
                                s_wait(1)

                                @pl.when(t < nbh - 1)
                                def _():
                                    idx_start(2 * t + 2, 0)
                        else:
                            s_wait(1 - p)
                        if j < IB - 1:
                            g_start(q, j + 1, 1 - p)
                        elif half == 0:
                            idx_wait(1)
                            g_start(1, 0, 0)
                        else:
                            @pl.when(t < nbh - 1)
                            def _():
                                idx_wait(0)
                                g_start(0, 0, 0)
                        s_start(q, j, p)

            s_wait(1)

        plsc.subcore_barrier()
        if rank1:
            pltpu.sync_copy(acc.at[pl.ds(base, RPT)],
                            out_hbm.at[c, pl.ds(base, RPT)])
        else:
            pltpu.sync_copy(
                acc.at[pl.ds(base, RPT)],
                out_hbm.at[pl.ds(base, RPT), pl.ds(c * DHALF, DHALF)])

    return prop


_sc_prop128 = _make_sc_prop(False)
_sc_prop1 = _make_sc_prop(True)


BLK = 1280
_GRID = NPAD // BLK


def _rowmask(i, rows=BLK):
    r = i * rows + lax.broadcasted_iota(jnp.int32, (rows, 1), 0)
    return r < N


def _deg_to_scale(dref, i, rows=BLK):
    d = dref[0] + dref[1]
    return jnp.where(_rowmask(i, rows), lax.rsqrt(jnp.maximum(d, 1.0)), 0.0)


def _dot(x, w):
    return jnp.dot(x, w, preferred_element_type=jnp.float32,
                   precision=lax.Precision.HIGHEST)


def _deg_spec():
    return pl.BlockSpec((2, BLK, 1), lambda i: (0, i, 0))


def _feat_spec():
    return pl.BlockSpec((BLK, NFEAT), lambda i: (i, 0))


def _half_spec():
    return pl.BlockSpec((BLK, DHALF), lambda i: (i, 0))


def _tc_linear(x, W, brow):
    def body(x_ref, w_ref, b_ref, o_ref):
        o_ref[...] = _dot(x_ref[...], w_ref[...]) + b_ref[...]

    return pl.pallas_call(
        body,
        out_shape=jax.ShapeDtypeStruct((NPAD, NFEAT), jnp.float32),
        grid=(_GRID,),
        in_specs=[_feat_spec(),
                  pl.BlockSpec((NFEAT, NFEAT), lambda i: (0, 0)),
                  pl.BlockSpec((1, NFEAT), lambda i: (0, 0))],
        out_specs=_feat_spec(),
    )(x, W, brow)


def _tc_scale(t, dsrc):
    def body(t_ref, ds_ref, o_ref):
        a = _deg_to_scale(ds_ref, pl.program_id(0))
        o_ref[...] = a * t_ref[...]

    return pl.pallas_call(
        body,
        out_shape=jax.ShapeDtypeStruct((NPAD, NFEAT), jnp.float32),
        grid=(_GRID,),
        in_specs=[_feat_spec(), _deg_spec()],
        out_specs=_feat_spec(),
    )(t, dsrc)


def _tc_combine1(p, dsrc, ddst, W, brow):
    def body(p_ref, ds_ref, dd_ref, w_ref, b_ref, o_ref):
        i = pl.program_id(0)
        b = _deg_to_scale(dd_ref, i)
        a = _deg_to_scale(ds_ref, i)
        hpre = p_ref[...] * b
        h = jnp.where(hpre >= 0, hpre, LEAK * hpre)
        o_ref[...] = a * (_dot(h, w_ref[...]) + b_ref[...])

    return pl.pallas_call(
        body,
        out_shape=jax.ShapeDtypeStruct((NPAD, NFEAT), jnp.float32),
        grid=(_GRID,),
        in_specs=[_feat_spec(),
                  _deg_spec(), _deg_spec(),
                  pl.BlockSpec((NFEAT, NFEAT), lambda i: (0, 0)),
                  pl.BlockSpec((1, NFEAT), lambda i: (0, 0))],
        out_specs=_feat_spec(),
    )(p, dsrc, ddst, W, brow)


def _tc_combine2(p, dsrc, ddst, w2row, b2s):
    def body(p_ref, ds_ref, dd_ref, w_ref, b_ref, h_ref, g_ref):
        i = pl.program_id(0)
        b = _deg_to_scale(dd_ref, i)
        a = _deg_to_scale(ds_ref, i)
        hpre = p_ref[...] * b
        h = jnp.where(hpre >= 0, hpre, LEAK * hpre)
        h_ref[...] = h
        sv = jnp.sum(h * w_ref[...], axis=1, keepdims=True) + b_ref[0, 0]
        g_ref[...] = a * sv

    return pl.pallas_call(
        body,
        out_shape=(jax.ShapeDtypeStruct((NPAD, NFEAT), jnp.float32),
                   jax.ShapeDtypeStruct((NPAD, 1), jnp.float32)),
        grid=(_GRID,),
        in_specs=[_feat_spec(),
                  _deg_spec(), _deg_spec(),
                  pl.BlockSpec((1, NFEAT), lambda i: (0, 0)),
                  pl.BlockSpec((1, 1), lambda i: (0, 0))],
        out_specs=(_feat_spec(),
                   pl.BlockSpec((BLK, 1), lambda i: (i, 0))),
    )(p, dsrc, ddst, w2row, b2s)


def _tc_softmax(p, ddst):
    def body(p_ref, dd_ref, o_ref):
        b = _deg_to_scale(dd_ref, 0, NPAD)
        l = b * (p_ref[0] + p_ref[1])
        mask = _rowmask(0, NPAD)
        lm = jnp.where(mask, l, -1e30)
        m = jnp.max(lm)
        e = jnp.where(mask, jnp.exp(lm - m), 0.0)
        o_ref[...] = e / jnp.sum(e)

    return pl.pallas_call(
        body,
        out_shape=jax.ShapeDtypeStruct((NPAD, 1), jnp.float32),
        grid=(1,),
        in_specs=[pl.BlockSpec((2, NPAD, 1), lambda i: (0, 0, 0)),
                  pl.BlockSpec((2, NPAD, 1), lambda i: (0, 0, 0))],
        out_specs=pl.BlockSpec((NPAD, 1), lambda i: (0, 0)),
    )(p, ddst)


def kernel(x, adj, W0, b0, W1, b1, W2, b2):
    src = adj[0].astype(jnp.int32)
    dst = adj[1].astype(jnp.int32)
    pad = jnp.full((EPAD - E,), N, jnp.int32)
    sp = jnp.concatenate([src, pad]).reshape(TOTCH, CHUNK)
    dp = jnp.concatenate([dst, pad]).reshape(TOTCH, CHUNK)
    sd3 = jnp.stack([sp, dp])
    ds3 = jnp.stack([dp, sp])
    xp = jnp.zeros((NPAD, NFEAT), jnp.float32).at[:N].set(x)
    ones1 = jnp.ones((NPAD,), jnp.float32)

    dsrc = _sc_prop1(ones1, ds3).reshape(NC, NPAD, 1)
    t0 = _tc_linear(xp, W0, b0.reshape(1, -1))
    ddst = _sc_prop1(ones1, sd3).reshape(NC, NPAD, 1)
    g0 = _tc_scale(t0, dsrc)
    p1 = _sc_prop128(g0, sd3)
    g1 = _tc_combine1(p1, dsrc, ddst, W1, b1.reshape(1, -1))
    p2 = _sc_prop128(g1, sd3)
    h2, g2 = _tc_combine2(p2, dsrc, ddst,
                          W2.reshape(1, -1), b2.reshape(1, 1))
    p3 = _sc_prop1(g2.reshape(NPAD), sd3).reshape(NC, NPAD, 1)
    w = _tc_softmax(p3, ddst)
    return (w[:N], h2[:N])


# speedup vs baseline: 3.6532x; 1.1868x over previous
import functools

import jax
import jax.numpy as jnp
from jax import lax
from jax.experimental import pallas as pl
from jax.experimental.pallas import tpu as pltpu
from jax.experimental.pallas import tpu_sc as plsc

N = 10000
NFEAT = 128
NPAD = 10240
E = 320000
NC = 2
NS = 16
NW = NC * NS
CHUNK = 128
IB = 8
NBUF = 4
EPAD = 327680
TOTCH = EPAD // CHUNK
RPT = NPAD // NS
LEAK = 0.01
DHALF = NFEAT // 2

_MESH = plsc.VectorSubcoreMesh(core_axis_name="c", subcore_axis_name="s")


def _fill_zero(ref, nrows, ncols):
    vec = jnp.zeros((16,), jnp.float32)

    @pl.loop(0, nrows)
    def _(r):
        for k in range(ncols // 16):
            ref[r, pl.ds(k * 16, 16)] = vec


def _fill_zero1(ref, n):
    vec = jnp.zeros((16,), jnp.float32)
    for k in range(n // 16):
        ref[pl.ds(k * 16, 16)] = vec


def _make_sc_prop(rank1):
    nbuf = 4 if rank1 else 2
    bufshape = (CHUNK,) if rank1 else (CHUNK, DHALF)
    nch = TOTCH // NW if rank1 else TOTCH // NS
    nbh = nch // (2 * IB)

    scratch = [
        pltpu.VMEM((2, IB, CHUNK), jnp.int32),
        pltpu.VMEM((2, IB, CHUNK), jnp.int32),
    ]
    scratch += [pltpu.VMEM(bufshape, jnp.float32)] * nbuf
    if rank1:
        scratch += [pltpu.VMEM_SHARED((NPAD,), jnp.float32)]
    else:
        scratch += [pltpu.VMEM_SHARED((NPAD, DHALF), jnp.float32),
                    pltpu.VMEM_SHARED((NPAD, DHALF), jnp.float32)]
    scratch += [pltpu.SemaphoreType.DMA] * (2 * nbuf + 2)

    if rank1:
        out_type = jax.ShapeDtypeStruct((NC, NPAD), jnp.float32)
        cp = None
    else:
        out_type = jax.ShapeDtypeStruct((NPAD, NFEAT), jnp.float32)
        cp = pltpu.CompilerParams(use_tc_tiling_on_sc=False)

    @functools.partial(pl.kernel, out_type=out_type, mesh=_MESH,
                       scratch_types=scratch, compiler_params=cp)
    def prop(*refs):
        if rank1:
            (h_hbm, sd_hbm, out_hbm, sidx, didx, r0, r1, r2, r3, acc,
             g0, g1, g2, g3, s0, s1, s2, s3, isem0, isem1) = refs
            rows, gsem, ssem = (r0, r1, r2, r3), (g0, g1, g2, g3), (s0, s1, s2, s3)
            isem = (isem0, isem1)
            tab = h_hbm
        else:
            (h_hbm, sd_hbm, out_hbm, sidx, didx,
             r0, r1, tab, acc, g0, g1, s0, s1, isem0, isem1) = refs
            rows, gsem, ssem = (r0, r1), (g0, g1), (s0, s1)
            isem = (isem0, isem1)

        c = lax.axis_index("c")
        s = lax.axis_index("s")
        base = s * RPT
        cb = ((c * NS + s) if rank1 else s) * nch

        def idx_start(b, q):
            pltpu.make_async_copy(sd_hbm.at[0, pl.ds(cb + b * IB, IB)],
                                  sidx.at[q], isem[q]).start()
            pltpu.make_async_copy(sd_hbm.at[1, pl.ds(cb + b * IB, IB)],
                                  didx.at[q], isem[q]).start()

        def idx_wait(q):
            pltpu.make_async_copy(sd_hbm.at[0, pl.ds(0, IB)],
                                  sidx.at[q], isem[q]).wait()
            pltpu.make_async_copy(sd_hbm.at[1, pl.ds(0, IB)],
                                  didx.at[q], isem[q]).wait()

        def g_start(q, j, p):
            pltpu.make_async_copy(tab.at[sidx.at[q, j]], rows[p],
                                  gsem[p]).start()

        def g_wait(p):
            pltpu.make_async_copy(tab.at[sidx.at[0, 0]], rows[p],
                                  gsem[p]).wait()

        def s_start(q, j, p):
            pltpu.make_async_copy(rows[p], acc.at[didx.at[q, j]],
                                  ssem[p]).start(add=True)

        def s_wait(p):
            pltpu.make_async_copy(rows[p], acc.at[didx.at[0, 0]],
                                  ssem[p]).wait()

        if rank1:
            _fill_zero1(r0, CHUNK)
        else:
            pltpu.sync_copy(
                h_hbm.at[pl.ds(base, RPT), pl.ds(c * DHALF, DHALF)],
                tab.at[pl.ds(base, RPT)])
            _fill_zero(r0, CHUNK, DHALF)
        for k in range(RPT // CHUNK):
            pltpu.sync_copy(r0, acc.at[pl.ds(base + k * CHUNK, CHUNK)])
        plsc.subcore_barrier()

        pltpu.sync_copy(sd_hbm.at[0, pl.ds(cb, IB)], sidx.at[0])
        pltpu.sync_copy(sd_hbm.at[1, pl.ds(cb, IB)], didx.at[0])
        idx_start(1, 1)
        g_start(0, 0, 0)
        if rank1:
            g_start(0, 1, 1)

        if rank1:
            @pl.loop(0, nbh)
            def _(t):
                for half in range(2):
                    q = half
                    for j in range(IB):
                        p = j % nbuf
                        np_ = (j + 2) % nbuf
                        g_wait(p)
                        if half == 0 and j < 2:
                            @pl.when(t > 0)
                            def _():
                                s_wait(np_)
                                if j == 1:
                                    idx_start(2 * t + 1, 1)
                        else:
                            s_wait(np_)
                            if half == 1 and j == 1:
                                @pl.when(t < nbh - 1)
                                def _():
                                    idx_start(2 * t + 2, 0)
                        if j == IB - 2:
                            if half == 0:
                                idx_wait(1)
                                g_start(1, 0, np_)
                            else:
                                @pl.when(t < nbh - 1)
                                def _():
                                    idx_wait(0)
                                    g_start(0, 0, np_)
                        elif j == IB - 1:
                            if half == 0:
                                g_start(1, 1, np_)
                            else:
                                @pl.when(t < nbh - 1)
                                def _():
                                    g_start(0, 1, np_)
                        else:
                            g_start(q, j + 2, np_)
                        s_start(q, j, p)

            s_wait((nch - 2) % nbuf)
            s_wait((nch - 1) % nbuf)
        else:
            @pl.loop(0, nbh)
            def _(t):
                for half in range(2):
                    q = half
                    for j in range(IB):
                        p = j % 2
                        g_wait(p)
                        if j == 0:
                            if half == 0:
                                @pl.when(t > 0)
                                def _():
                                    s_wait(1)
                                    idx_start(2 * t + 1, 1)
                            else:
                                s_wait(1)

                                @pl.when(t < nbh - 1)
                                def _():
                                    idx_start(2 * t + 2, 0)
                        else:
                            s_wait(1 - p)
                        if j < IB - 1:
                            g_start(q, j + 1, 1 - p)
                        elif half == 0:
                            idx_wait(1)
                            g_start(1, 0, 0)
                        else:
                            @pl.when(t < nbh - 1)
                            def _():
                                idx_wait(0)
                                g_start(0, 0, 0)
                        s_start(q, j, p)

            s_wait(1)

        plsc.subcore_barrier()
        if rank1:
            pltpu.sync_copy(acc.at[pl.ds(base, RPT)],
                            out_hbm.at[c, pl.ds(base, RPT)])
        else:
            pltpu.sync_copy(
                acc.at[pl.ds(base, RPT)],
                out_hbm.at[pl.ds(base, RPT), pl.ds(c * DHALF, DHALF)])

    return prop


_sc_prop128 = _make_sc_prop(False)
_sc_prop1 = _make_sc_prop(True)


def _fill_one1(ref, n):
    vec = jnp.full((16,), 1.0, jnp.float32)
    for k in range(n // 16):
        ref[pl.ds(k * 16, 16)] = vec


_HIST_NCH = TOTCH // NW
_HIST_NBH = _HIST_NCH // (2 * IB)


@functools.partial(
    pl.kernel,
    out_type=jax.ShapeDtypeStruct((NC, NPAD), jnp.float32),
    mesh=_MESH,
    scratch_types=[
        pltpu.VMEM((2, IB, CHUNK), jnp.int32),
        pltpu.VMEM((CHUNK,), jnp.float32),
        pltpu.VMEM_SHARED((NPAD,), jnp.float32),
        pltpu.SemaphoreType.DMA,
        pltpu.SemaphoreType.DMA,
        pltpu.SemaphoreType.DMA,
        pltpu.SemaphoreType.DMA,
        pltpu.SemaphoreType.DMA,
        pltpu.SemaphoreType.DMA,
    ],
)
def _sc_hist(ix_hbm, out_hbm, didx, r0, acc, s0, s1, s2, s3, isem0, isem1):
    c = lax.axis_index("c")
    s = lax.axis_index("s")
    base = s * RPT
    cb = (c * NS + s) * _HIST_NCH
    ssem = (s0, s1, s2, s3)
    isem = (isem0, isem1)

    def idx_start(b, q):
        pltpu.make_async_copy(ix_hbm.at[pl.ds(cb + b * IB, IB)],
                              didx.at[q], isem[q]).start()

    def idx_wait(q):
        pltpu.make_async_copy(ix_hbm.at[pl.ds(0, IB)],
                              didx.at[q], isem[q]).wait()

    def s_start(q, j, p):
        pltpu.make_async_copy(r0, acc.at[didx.at[q, j]],
                              ssem[p]).start(add=True)

    def s_wait(p):
        pltpu.make_async_copy(r0, acc.at[didx.at[0, 0]], ssem[p]).wait()

    _fill_zero1(r0, CHUNK)
    for k in range(RPT // CHUNK):
        pltpu.sync_copy(r0, acc.at[pl.ds(base + k * CHUNK, CHUNK)])
    _fill_one1(r0, CHUNK)
    plsc.subcore_barrier()

    pltpu.sync_copy(ix_hbm.at[pl.ds(cb, IB)], didx.at[0])
    idx_start(1, 1)

    @pl.loop(0, _HIST_NBH)
    def _(t):
        for half in range(2):
            q = half
            for j in range(IB):
                p = j % 4
                if half == 0 and j < 4:
                    @pl.when(t > 0)
                    def _():
                        s_wait(p)
                        if j == 1:
                            idx_start(2 * t + 1, 1)
                else:
                    s_wait(p)
                    if half == 1 and j == 1:
                        @pl.when(t < _HIST_NBH - 1)
                        def _():
                            idx_start(2 * t + 2, 0)
                if j == IB - 1:
                    if half == 0:
                        idx_wait(1)
                    else:
                        @pl.when(t < _HIST_NBH - 1)
                        def _():
                            idx_wait(0)
                s_start(q, j, p)

    for p in range(4):
        s_wait(p)
    plsc.subcore_barrier()
    pltpu.sync_copy(acc.at[pl.ds(base, RPT)],
                    out_hbm.at[c, pl.ds(base, RPT)])


BLK = 1280
_GRID = NPAD // BLK


def _rowmask(i, rows=BLK):
    r = i * rows + lax.broadcasted_iota(jnp.int32, (rows, 1), 0)
    return r < N


def _deg_to_scale(dref, i, rows=BLK):
    d = dref[0] + dref[1]
    return jnp.where(_rowmask(i, rows), lax.rsqrt(jnp.maximum(d, 1.0)), 0.0)


def _dot(x, w):
    return jnp.dot(x, w, preferred_element_type=jnp.float32,
                   precision=lax.Precision.HIGHEST)


def _deg_spec():
    return pl.BlockSpec((2, BLK, 1), lambda i: (0, i, 0))


def _feat_spec():
    return pl.BlockSpec((BLK, NFEAT), lambda i: (i, 0))


def _half_spec():
    return pl.BlockSpec((BLK, DHALF), lambda i: (i, 0))


def _tc_linear(x, W, brow):
    def body(x_ref, w_ref, b_ref, o_ref):
        o_ref[...] = _dot(x_ref[...], w_ref[...]) + b_ref[...]

    return pl.pallas_call(
        body,
        out_shape=jax.ShapeDtypeStruct((NPAD, NFEAT), jnp.float32),
        grid=(_GRID,),
        in_specs=[_feat_spec(),
                  pl.BlockSpec((NFEAT, NFEAT), lambda i: (0, 0)),
                  pl.BlockSpec((1, NFEAT), lambda i: (0, 0))],
        out_specs=_feat_spec(),
    )(x, W, brow)


def _tc_scale(t, dsrc):
    def body(t_ref, ds_ref, o_ref):
        a = _deg_to_scale(ds_ref, pl.program_id(0))
        o_ref[...] = a * t_ref[...]

    return pl.pallas_call(
        body,
        out_shape=jax.ShapeDtypeStruct((NPAD, NFEAT), jnp.float32),
        grid=(_GRID,),
        in_specs=[_feat_spec(), _deg_spec()],
        out_specs=_feat_spec(),
    )(t, dsrc)


def _tc_combine1(p, dsrc, ddst, W, brow):
    def body(p_ref, ds_ref, dd_ref, w_ref, b_ref, o_ref):
        i = pl.program_id(0)
        b = _deg_to_scale(dd_ref, i)
        a = _deg_to_scale(ds_ref, i)
        hpre = p_ref[...] * b
        h = jnp.where(hpre >= 0, hpre, LEAK * hpre)
        o_ref[...] = a * (_dot(h, w_ref[...]) + b_ref[...])

    return pl.pallas_call(
        body,
        out_shape=jax.ShapeDtypeStruct((NPAD, NFEAT), jnp.float32),
        grid=(_GRID,),
        in_specs=[_feat_spec(),
                  _deg_spec(), _deg_spec(),
                  pl.BlockSpec((NFEAT, NFEAT), lambda i: (0, 0)),
                  pl.BlockSpec((1, NFEAT), lambda i: (0, 0))],
        out_specs=_feat_spec(),
    )(p, dsrc, ddst, W, brow)


def _tc_combine2(p, dsrc, ddst, w2row, b2s):
    def body(p_ref, ds_ref, dd_ref, w_ref, b_ref, h_ref, g_ref):
        i = pl.program_id(0)
        b = _deg_to_scale(dd_ref, i)
        a = _deg_to_scale(ds_ref, i)
        hpre = p_ref[...] * b
        h = jnp.where(hpre >= 0, hpre, LEAK * hpre)
        h_ref[...] = h
        sv = jnp.sum(h * w_ref[...], axis=1, keepdims=True) + b_ref[0, 0]
        g_ref[...] = a * sv

    return pl.pallas_call(
        body,
        out_shape=(jax.ShapeDtypeStruct((NPAD, NFEAT), jnp.float32),
                   jax.ShapeDtypeStruct((NPAD, 1), jnp.float32)),
        grid=(_GRID,),
        in_specs=[_feat_spec(),
                  _deg_spec(), _deg_spec(),
                  pl.BlockSpec((1, NFEAT), lambda i: (0, 0)),
                  pl.BlockSpec((1, 1), lambda i: (0, 0))],
        out_specs=(_feat_spec(),
                   pl.BlockSpec((BLK, 1), lambda i: (i, 0))),
    )(p, dsrc, ddst, w2row, b2s)


def _tc_softmax(p, ddst):
    def body(p_ref, dd_ref, o_ref):
        b = _deg_to_scale(dd_ref, 0, NPAD)
        l = b * (p_ref[0] + p_ref[1])
        mask = _rowmask(0, NPAD)
        lm = jnp.where(mask, l, -1e30)
        m = jnp.max(lm)
        e = jnp.where(mask, jnp.exp(lm - m), 0.0)
        o_ref[...] = e / jnp.sum(e)

    return pl.pallas_call(
        body,
        out_shape=jax.ShapeDtypeStruct((NPAD, 1), jnp.float32),
        grid=(1,),
        in_specs=[pl.BlockSpec((2, NPAD, 1), lambda i: (0, 0, 0)),
                  pl.BlockSpec((2, NPAD, 1), lambda i: (0, 0, 0))],
        out_specs=pl.BlockSpec((NPAD, 1), lambda i: (0, 0)),
    )(p, ddst)


def kernel(x, adj, W0, b0, W1, b1, W2, b2):
    src = adj[0].astype(jnp.int32)
    dst = adj[1].astype(jnp.int32)
    pad = jnp.full((EPAD - E,), N, jnp.int32)
    sp = jnp.concatenate([src, pad]).reshape(TOTCH, CHUNK)
    dp = jnp.concatenate([dst, pad]).reshape(TOTCH, CHUNK)
    sd3 = jnp.stack([sp, dp])
    xp = jnp.zeros((NPAD, NFEAT), jnp.float32).at[:N].set(x)

    dsrc = _sc_hist(sp).reshape(NC, NPAD, 1)
    t0 = _tc_linear(xp, W0, b0.reshape(1, -1))
    ddst = _sc_hist(dp).reshape(NC, NPAD, 1)
    g0 = _tc_scale(t0, dsrc)
    p1 = _sc_prop128(g0, sd3)
    g1 = _tc_combine1(p1, dsrc, ddst, W1, b1.reshape(1, -1))
    p2 = _sc_prop128(g1, sd3)
    h2, g2 = _tc_combine2(p2, dsrc, ddst,
                          W2.reshape(1, -1), b2.reshape(1, 1))
    p3 = _sc_prop1(g2.reshape(NPAD), sd3).reshape(NC, NPAD, 1)
    w = _tc_softmax(p3, ddst)
    return (w[:N], h2[:N])
